# baseline (speedup 1.0000x reference)
import functools

import jax
import jax.numpy as jnp
from jax import lax
from jax.experimental import pallas as pl
from jax.experimental.pallas import tpu as pltpu
from jax.experimental.pallas import tpu_sc as plsc

_NC, _NS, _L = 2, 16, 16
_NW = _NC * _NS
_B = 16384
_BPW = _B // _NW
_CHUNK = 128
_NCHUNK = _BPW // _CHUNK
_D = 32


def _body(u_idx_hbm, p_idx_hbm, n_idx_hbm, utab_hbm, itab_hbm,
          x_out, ue_out, pe_out, ne_out,
          u_idx, p_idx, n_idx, u_rows, p_rows, n_rows, x_v, gsem, wsem):
    wid = lax.axis_index("s") * _NC + lax.axis_index("c")
    pltpu.sync_copy(u_idx_hbm.at[wid], u_idx)
    pltpu.sync_copy(p_idx_hbm.at[wid], p_idx)
    pltpu.sync_copy(n_idx_hbm.at[wid], n_idx)

    gathers = []
    for c in range(_NCHUNK):
        sl = pl.ds(c * _CHUNK, _CHUNK)
        gathers.append(pltpu.async_copy(utab_hbm.at[u_idx.at[c]], u_rows.at[sl], gsem))
        gathers.append(pltpu.async_copy(itab_hbm.at[p_idx.at[c]], p_rows.at[sl], gsem))
        gathers.append(pltpu.async_copy(itab_hbm.at[n_idx.at[c]], n_rows.at[sl], gsem))
    for g in gathers:
        g.wait()

    base = wid * _BPW
    w_u = pltpu.async_copy(u_rows, ue_out.at[pl.ds(base, _BPW)], wsem)
    w_p = pltpu.async_copy(p_rows, pe_out.at[pl.ds(base, _BPW)], wsem)
    w_n = pltpu.async_copy(n_rows, ne_out.at[pl.ds(base, _BPW)], wsem)

    iot = lax.iota(jnp.int32, _L)

    def blk_body(blk, _):
        ridx = iot + blk * _L

        def j_body(j, acc):
            col = jnp.zeros((_L,), jnp.int32) + j
            u = plsc.load_gather(u_rows, [ridx, col])
            p = plsc.load_gather(p_rows, [ridx, col])
            n = plsc.load_gather(n_rows, [ridx, col])
            return acc + u * (p - n)

        acc = lax.fori_loop(0, _D, j_body, jnp.zeros((_L,), jnp.float32),
                            unroll=4)
        x_v[pl.ds(blk * _L, _L)] = acc
        return 0

    lax.fori_loop(0, _BPW // _L, blk_body, 0)

    pltpu.sync_copy(x_v, x_out.at[pl.ds(base, _BPW)])
    w_u.wait()
    w_p.wait()
    w_n.wait()


_RB = 8192


def _repack_body(ut_ref, it_ref, uo_ref, io_ref):
    uo_ref[...] = ut_ref[...].T
    io_ref[...] = it_ref[...].T


def _repack(user_table, item_table):
    n = user_table.shape[0]
    grid = (n + _RB - 1) // _RB
    return pl.pallas_call(
        _repack_body,
        grid=(grid,),
        in_specs=[
            pl.BlockSpec((_D, _RB), lambda i: (0, i)),
            pl.BlockSpec((_D, _RB), lambda i: (0, i)),
        ],
        out_specs=[
            pl.BlockSpec((_RB, _D), lambda i: (i, 0)),
            pl.BlockSpec((_RB, _D), lambda i: (i, 0)),
        ],
        out_shape=[
            jax.ShapeDtypeStruct((n, _D), jnp.float32),
            jax.ShapeDtypeStruct((n, _D), jnp.float32),
        ],
        compiler_params=pltpu.CompilerParams(
            dimension_semantics=("arbitrary",)),
    )(user_table.T, item_table.T)


def kernel(user, pos_item, neg_item, user_table, item_table):
    user_table, item_table = _repack(user_table, item_table)
    u = user.astype(jnp.int32).reshape(_NW, _NCHUNK, _CHUNK)
    p = pos_item.astype(jnp.int32).reshape(_NW, _NCHUNK, _CHUNK)
    n = neg_item.astype(jnp.int32).reshape(_NW, _NCHUNK, _CHUNK)

    mesh = plsc.VectorSubcoreMesh(core_axis_name="c", subcore_axis_name="s",
                                  num_cores=_NC, num_subcores=_NS)
    f = pl.kernel(
        _body,
        out_type=(
            jax.ShapeDtypeStruct((_B,), jnp.float32),
            jax.ShapeDtypeStruct((_B, _D), jnp.float32),
            jax.ShapeDtypeStruct((_B, _D), jnp.float32),
            jax.ShapeDtypeStruct((_B, _D), jnp.float32),
        ),
        mesh=mesh,
        compiler_params=pltpu.CompilerParams(
            use_tc_tiling_on_sc=False, needs_layout_passes=False),
        scratch_types=[
            pltpu.VMEM((_NCHUNK, _CHUNK), jnp.int32),
            pltpu.VMEM((_NCHUNK, _CHUNK), jnp.int32),
            pltpu.VMEM((_NCHUNK, _CHUNK), jnp.int32),
            pltpu.VMEM((_BPW, _D), jnp.float32),
            pltpu.VMEM((_BPW, _D), jnp.float32),
            pltpu.VMEM((_BPW, _D), jnp.float32),
            pltpu.VMEM((_BPW,), jnp.float32),
            pltpu.SemaphoreType.DMA,
            pltpu.SemaphoreType.DMA,
        ],
    )
    x, ue, pe, ne = f(u, p, n, user_table, item_table)
    return (x, ue, pe, ne)


# speedup vs baseline: 1.2308x; 1.2308x over previous
import functools

import jax
import jax.numpy as jnp
from jax import lax
from jax.experimental import pallas as pl
from jax.experimental.pallas import tpu as pltpu
from jax.experimental.pallas import tpu_sc as plsc

_NC, _NS, _L = 2, 16, 16
_NW = _NC * _NS
_B = 16384
_BPW = _B // _NW
_CHUNK = 128
_NCHUNK = _BPW // _CHUNK
_D = 32


def _body(u_idx_hbm, p_idx_hbm, n_idx_hbm, utab_hbm, itab_hbm,
          x_out, ue_out, pe_out, ne_out,
          u_idx, p_idx, n_idx, u_rows, p_rows, n_rows,
          u_t, p_t, n_t, x_v, gsem, wsem):
    wid = lax.axis_index("s") * _NC + lax.axis_index("c")
    base = wid * _BPW
    pltpu.sync_copy(u_idx_hbm.at[wid], u_idx)
    pltpu.sync_copy(p_idx_hbm.at[wid], p_idx)
    pltpu.sync_copy(n_idx_hbm.at[wid], n_idx)

    gathers = []
    for c in range(_NCHUNK):
        sl = pl.ds(c * _CHUNK, _CHUNK)
        gathers.append(pltpu.async_copy(utab_hbm.at[u_idx.at[c]], u_rows.at[sl], gsem))
        gathers.append(pltpu.async_copy(itab_hbm.at[p_idx.at[c]], p_rows.at[sl], gsem))
        gathers.append(pltpu.async_copy(itab_hbm.at[n_idx.at[c]], n_rows.at[sl], gsem))
    for g in gathers:
        g.wait()

    iot = lax.iota(jnp.int32, _L)

    def blk_body(blk, _):
        ridx = iot + blk * _L
        off = blk * _L

        def j_body(j, acc):
            col = jnp.zeros((_L,), jnp.int32) + j
            u = plsc.load_gather(u_rows, [ridx, col])
            p = plsc.load_gather(p_rows, [ridx, col])
            n = plsc.load_gather(n_rows, [ridx, col])
            u_t[j, pl.ds(off, _L)] = u
            p_t[j, pl.ds(off, _L)] = p
            n_t[j, pl.ds(off, _L)] = n
            return acc + u * (p - n)

        acc = lax.fori_loop(0, _D, j_body, jnp.zeros((_L,), jnp.float32),
                            unroll=4)
        x_v[pl.ds(off, _L)] = acc
        return 0

    lax.fori_loop(0, _BPW // _L, blk_body, 0)

    w_u = pltpu.async_copy(u_t, ue_out.at[:, pl.ds(base, _BPW)], wsem)
    w_p = pltpu.async_copy(p_t, pe_out.at[:, pl.ds(base, _BPW)], wsem)
    w_n = pltpu.async_copy(n_t, ne_out.at[:, pl.ds(base, _BPW)], wsem)
    pltpu.sync_copy(x_v, x_out.at[pl.ds(base, _BPW)])
    w_u.wait()
    w_p.wait()
    w_n.wait()


def kernel(user, pos_item, neg_item, user_table, item_table):
    u = user.astype(jnp.int32).reshape(_NW, _NCHUNK, _CHUNK)
    p = pos_item.astype(jnp.int32).reshape(_NW, _NCHUNK, _CHUNK)
    n = neg_item.astype(jnp.int32).reshape(_NW, _NCHUNK, _CHUNK)

    mesh = plsc.VectorSubcoreMesh(core_axis_name="c", subcore_axis_name="s",
                                  num_cores=_NC, num_subcores=_NS)
    f = pl.kernel(
        _body,
        out_type=(
            jax.ShapeDtypeStruct((_B,), jnp.float32),
            jax.ShapeDtypeStruct((_D, _B), jnp.float32),
            jax.ShapeDtypeStruct((_D, _B), jnp.float32),
            jax.ShapeDtypeStruct((_D, _B), jnp.float32),
        ),
        mesh=mesh,
        compiler_params=pltpu.CompilerParams(
            use_tc_tiling_on_sc=False, needs_layout_passes=False),
        scratch_types=[
            pltpu.VMEM((_NCHUNK, _CHUNK), jnp.int32),
            pltpu.VMEM((_NCHUNK, _CHUNK), jnp.int32),
            pltpu.VMEM((_NCHUNK, _CHUNK), jnp.int32),
            pltpu.VMEM((_BPW, _D), jnp.float32),
            pltpu.VMEM((_BPW, _D), jnp.float32),
            pltpu.VMEM((_BPW, _D), jnp.float32),
            pltpu.VMEM((_D, _BPW), jnp.float32),
            pltpu.VMEM((_D, _BPW), jnp.float32),
            pltpu.VMEM((_D, _BPW), jnp.float32),
            pltpu.VMEM((_BPW,), jnp.float32),
            pltpu.SemaphoreType.DMA,
            pltpu.SemaphoreType.DMA,
        ],
    )
    x, ue_t, pe_t, ne_t = f(u, p, n, user_table, item_table)
    return (x, ue_t.T, pe_t.T, ne_t.T)
